# baseline (device time: 121080 ns/iter reference)
import jax
import jax.numpy as jnp
from jax import lax
from jax.experimental import pallas as pl
from jax.experimental.pallas import tpu as pltpu


def kernel(ids, E):
    T = ids.shape[0]
    V, D = E.shape
    TY = T // 2
    C = 64
    TC = TY // C

    my_x = lax.axis_index("x")
    my_y = lax.axis_index("y")

    ids_y = lax.dynamic_slice(ids, (my_y * TY,), (TY,))
    idx = ids_y - my_x * V
    idxc = jnp.clip(idx, 0, V - 1)
    idx2d = idx.reshape(TY, 1)

    def body(idx_smem_ref, idx_ref, E_ref, out_ref,
             stage_ref, partial_ref, xrecv_ref,
             g_sems, x_send_sems, x_recv_sems, y_send_sems, y_recv_sems):
        mx = lax.axis_index("x")
        my = lax.axis_index("y")
        ox = 1 - mx
        oy = 1 - my

        def gather_issue(c):
            base = c * TC
            def issue(i, carry):
                row = idx_smem_ref[base + i]
                pltpu.make_async_copy(
                    E_ref.at[pl.ds(row, 1), :],
                    stage_ref.at[pl.ds(base + i, 1), :],
                    g_sems.at[c],
                ).start()
                return carry
            lax.fori_loop(0, TC, issue, 0, unroll=8)

        def gather_wait(c):
            def w(i, carry):
                pltpu.make_async_copy(
                    E_ref.at[pl.ds(0, 1), :],
                    stage_ref.at[pl.ds(0, 1), :],
                    g_sems.at[c],
                ).wait()
                return carry
            lax.fori_loop(0, TC, w, 0, unroll=8)

        gather_issue(0)
        gather_issue(1)
        gather_issue(2)

        barrier_sem = pltpu.get_barrier_semaphore()
        pl.semaphore_signal(barrier_sem, inc=1, device_id=(ox, my),
                            device_id_type=pl.DeviceIdType.MESH)
        pl.semaphore_signal(barrier_sem, inc=1, device_id=(mx, oy),
                            device_id_type=pl.DeviceIdType.MESH)
        pl.semaphore_wait(barrier_sem, 2)

        rdma_x = []
        rdma_y = []

        def reduce_and_forward(c):
            rdma_x[c].wait_recv()
            row0 = my * TY + c * TC
            out_ref[pl.ds(row0, TC), :] = (
                partial_ref[pl.ds(c * TC, TC), :]
                + xrecv_ref[pl.ds(c * TC, TC), :]
            )
            r = pltpu.make_async_remote_copy(
                src_ref=out_ref.at[pl.ds(row0, TC), :],
                dst_ref=out_ref.at[pl.ds(row0, TC), :],
                send_sem=y_send_sems.at[c],
                recv_sem=y_recv_sems.at[c],
                device_id=(mx, oy),
                device_id_type=pl.DeviceIdType.MESH,
            )
            r.start()
            rdma_y.append(r)

        for c in range(C):
            gather_wait(c)
            own_c = (idx_ref[pl.ds(c * TC, TC), :] >= 0) & (
                idx_ref[pl.ds(c * TC, TC), :] < V
            )
            partial_ref[pl.ds(c * TC, TC), :] = jnp.where(
                own_c, stage_ref[pl.ds(c * TC, TC), :], 0.0
            ).astype(jnp.bfloat16)
            r = pltpu.make_async_remote_copy(
                src_ref=partial_ref.at[pl.ds(c * TC, TC), :],
                dst_ref=xrecv_ref.at[pl.ds(c * TC, TC), :],
                send_sem=x_send_sems.at[c],
                recv_sem=x_recv_sems.at[c],
                device_id=(ox, my),
                device_id_type=pl.DeviceIdType.MESH,
            )
            r.start()
            rdma_x.append(r)
            if c + 3 < C:
                gather_issue(c + 3)
            if c >= 1:
                reduce_and_forward(c - 1)
        reduce_and_forward(C - 1)

        for c in range(C):
            rdma_x[c].wait_send()
            rdma_y[c].wait_send()
            rdma_y[c].wait_recv()

    out = pl.pallas_call(
        body,
        out_shape=jax.ShapeDtypeStruct((T, D), jnp.bfloat16),
        in_specs=[
            pl.BlockSpec(memory_space=pltpu.SMEM),
            pl.BlockSpec(memory_space=pltpu.VMEM),
            pl.BlockSpec(memory_space=pl.ANY),
        ],
        out_specs=pl.BlockSpec(memory_space=pltpu.VMEM),
        scratch_shapes=[
            pltpu.VMEM((TY, D), jnp.float32),
            pltpu.VMEM((TY, D), jnp.bfloat16),
            pltpu.VMEM((TY, D), jnp.bfloat16),
            pltpu.SemaphoreType.DMA((C,)),
            pltpu.SemaphoreType.DMA((C,)),
            pltpu.SemaphoreType.DMA((C,)),
            pltpu.SemaphoreType.DMA((C,)),
            pltpu.SemaphoreType.DMA((C,)),
        ],
        compiler_params=pltpu.CompilerParams(collective_id=0),
    )(idxc, idx2d, E)

    return out


# device time: 111181 ns/iter; 1.0890x vs baseline; 1.0890x over previous
import jax
import jax.numpy as jnp
from jax import lax
from jax.experimental import pallas as pl
from jax.experimental.pallas import tpu as pltpu


def kernel(ids, E):
    T = ids.shape[0]
    V, D = E.shape
    TY = T // 2
    C = 32
    TC = TY // C

    my_x = lax.axis_index("x")
    my_y = lax.axis_index("y")

    ids_y = lax.dynamic_slice(ids, (my_y * TY,), (TY,))
    idx = ids_y - my_x * V
    idxc = jnp.clip(idx, 0, V - 1)
    idx2d = idx.reshape(TY, 1)

    def body(idx_smem_ref, idx_ref, E_ref, out_ref,
             stage_ref, partial_ref, xrecv_ref,
             g_sems, x_send_sems, x_recv_sems, y_send_sems, y_recv_sems):
        mx = lax.axis_index("x")
        my = lax.axis_index("y")
        ox = 1 - mx
        oy = 1 - my

        def gather_issue(c):
            base = c * TC
            def issue(i, carry):
                row = idx_smem_ref[base + i]
                pltpu.make_async_copy(
                    E_ref.at[pl.ds(row, 1), :],
                    stage_ref.at[pl.ds(base + i, 1), :],
                    g_sems.at[c],
                ).start()
                return carry
            lax.fori_loop(0, TC, issue, 0, unroll=8)

        def gather_wait(c):
            def w(i, carry):
                pltpu.make_async_copy(
                    E_ref.at[pl.ds(0, 1), :],
                    stage_ref.at[pl.ds(0, 1), :],
                    g_sems.at[c],
                ).wait()
                return carry
            lax.fori_loop(0, TC, w, 0, unroll=8)

        gather_issue(0)
        gather_issue(1)
        gather_issue(2)
        gather_issue(3)

        barrier_sem = pltpu.get_barrier_semaphore()
        pl.semaphore_signal(barrier_sem, inc=1, device_id=(ox, my),
                            device_id_type=pl.DeviceIdType.MESH)
        pl.semaphore_signal(barrier_sem, inc=1, device_id=(mx, oy),
                            device_id_type=pl.DeviceIdType.MESH)
        pl.semaphore_wait(barrier_sem, 2)

        rdma_x = []
        rdma_y = []

        def reduce_and_forward(c):
            rdma_x[c].wait_recv()
            row0 = my * TY + c * TC
            out_ref[pl.ds(row0, TC), :] = (
                partial_ref[pl.ds(c * TC, TC), :]
                + xrecv_ref[pl.ds(c * TC, TC), :]
            )
            r = pltpu.make_async_remote_copy(
                src_ref=out_ref.at[pl.ds(row0, TC), :],
                dst_ref=out_ref.at[pl.ds(row0, TC), :],
                send_sem=y_send_sems.at[c],
                recv_sem=y_recv_sems.at[c],
                device_id=(mx, oy),
                device_id_type=pl.DeviceIdType.MESH,
            )
            r.start()
            rdma_y.append(r)

        for c in range(C):
            gather_wait(c)
            own_c = (idx_ref[pl.ds(c * TC, TC), :] >= 0) & (
                idx_ref[pl.ds(c * TC, TC), :] < V
            )
            partial_ref[pl.ds(c * TC, TC), :] = jnp.where(
                own_c, stage_ref[pl.ds(c * TC, TC), :], 0.0
            ).astype(jnp.bfloat16)
            r = pltpu.make_async_remote_copy(
                src_ref=partial_ref.at[pl.ds(c * TC, TC), :],
                dst_ref=xrecv_ref.at[pl.ds(c * TC, TC), :],
                send_sem=x_send_sems.at[c],
                recv_sem=x_recv_sems.at[c],
                device_id=(ox, my),
                device_id_type=pl.DeviceIdType.MESH,
            )
            r.start()
            rdma_x.append(r)
            if c + 4 < C:
                gather_issue(c + 4)
            if c >= 1:
                reduce_and_forward(c - 1)
        reduce_and_forward(C - 1)

        for c in range(C):
            rdma_x[c].wait_send()
            rdma_y[c].wait_send()
            rdma_y[c].wait_recv()

    out = pl.pallas_call(
        body,
        out_shape=jax.ShapeDtypeStruct((T, D), jnp.bfloat16),
        in_specs=[
            pl.BlockSpec(memory_space=pltpu.SMEM),
            pl.BlockSpec(memory_space=pltpu.VMEM),
            pl.BlockSpec(memory_space=pl.ANY),
        ],
        out_specs=pl.BlockSpec(memory_space=pltpu.VMEM),
        scratch_shapes=[
            pltpu.VMEM((TY, D), jnp.float32),
            pltpu.VMEM((TY, D), jnp.bfloat16),
            pltpu.VMEM((TY, D), jnp.bfloat16),
            pltpu.SemaphoreType.DMA((C,)),
            pltpu.SemaphoreType.DMA((C,)),
            pltpu.SemaphoreType.DMA((C,)),
            pltpu.SemaphoreType.DMA((C,)),
            pltpu.SemaphoreType.DMA((C,)),
        ],
        compiler_params=pltpu.CompilerParams(collective_id=0),
    )(idxc, idx2d, E)

    return out


# device time: 110084 ns/iter; 1.0999x vs baseline; 1.0100x over previous
import jax
import jax.numpy as jnp
from jax import lax
from jax.experimental import pallas as pl
from jax.experimental.pallas import tpu as pltpu


def kernel(ids, E):
    T = ids.shape[0]
    V, D = E.shape
    TY = T // 2
    C = 32
    TC = TY // C

    my_x = lax.axis_index("x")
    my_y = lax.axis_index("y")

    ids_y = lax.dynamic_slice(ids, (my_y * TY,), (TY,))
    idx = ids_y - my_x * V
    idxc = jnp.clip(idx, 0, V - 1)
    idx2d = idx.reshape(TY, 1)

    def body(idx_smem_ref, idx_ref, E_ref, out_ref,
             stage_ref, partial_ref, xrecv_ref,
             g_sems, x_send_sems, x_recv_sems, y_send_sems, y_recv_sems):
        mx = lax.axis_index("x")
        my = lax.axis_index("y")
        ox = 1 - mx
        oy = 1 - my

        def gather_issue(c):
            base = c * TC
            def issue(i, carry):
                row = idx_smem_ref[base + i]
                pltpu.make_async_copy(
                    E_ref.at[pl.ds(row, 1), :],
                    stage_ref.at[pl.ds(base + i, 1), :],
                    g_sems.at[c],
                ).start()
                return carry
            lax.fori_loop(0, TC, issue, 0, unroll=8)

        def gather_wait(c):
            def w(i, carry):
                pltpu.make_async_copy(
                    E_ref.at[pl.ds(0, 1), :],
                    stage_ref.at[pl.ds(0, 1), :],
                    g_sems.at[c],
                ).wait()
                return carry
            lax.fori_loop(0, TC, w, 0, unroll=8)

        gather_issue(0)
        gather_issue(1)
        gather_issue(2)

        barrier_sem = pltpu.get_barrier_semaphore()
        pl.semaphore_signal(barrier_sem, inc=1, device_id=(ox, my),
                            device_id_type=pl.DeviceIdType.MESH)
        pl.semaphore_signal(barrier_sem, inc=1, device_id=(mx, oy),
                            device_id_type=pl.DeviceIdType.MESH)
        pl.semaphore_wait(barrier_sem, 2)

        rdma_x = []
        rdma_y = []

        def reduce_and_forward(c):
            rdma_x[c].wait_recv()
            row0 = my * TY + c * TC
            out_ref[pl.ds(row0, TC), :] = (
                partial_ref[pl.ds(c * TC, TC), :]
                + xrecv_ref[pl.ds(c * TC, TC), :]
            )
            r = pltpu.make_async_remote_copy(
                src_ref=out_ref.at[pl.ds(row0, TC), :],
                dst_ref=out_ref.at[pl.ds(row0, TC), :],
                send_sem=y_send_sems.at[c],
                recv_sem=y_recv_sems.at[c],
                device_id=(mx, oy),
                device_id_type=pl.DeviceIdType.MESH,
            )
            r.start()
            rdma_y.append(r)

        for c in range(C):
            gather_wait(c)
            own_c = (idx_ref[pl.ds(c * TC, TC), :] >= 0) & (
                idx_ref[pl.ds(c * TC, TC), :] < V
            )
            partial_ref[pl.ds(c * TC, TC), :] = jnp.where(
                own_c, stage_ref[pl.ds(c * TC, TC), :], 0.0
            ).astype(jnp.bfloat16)
            r = pltpu.make_async_remote_copy(
                src_ref=partial_ref.at[pl.ds(c * TC, TC), :],
                dst_ref=xrecv_ref.at[pl.ds(c * TC, TC), :],
                send_sem=x_send_sems.at[c],
                recv_sem=x_recv_sems.at[c],
                device_id=(ox, my),
                device_id_type=pl.DeviceIdType.MESH,
            )
            r.start()
            rdma_x.append(r)
            if c + 3 < C:
                gather_issue(c + 3)
            if c >= 1:
                reduce_and_forward(c - 1)
        reduce_and_forward(C - 1)

        for c in range(C):
            rdma_x[c].wait_send()
            rdma_y[c].wait_send()
            rdma_y[c].wait_recv()

    out = pl.pallas_call(
        body,
        out_shape=jax.ShapeDtypeStruct((T, D), jnp.bfloat16),
        in_specs=[
            pl.BlockSpec(memory_space=pltpu.SMEM),
            pl.BlockSpec(memory_space=pltpu.VMEM),
            pl.BlockSpec(memory_space=pl.ANY),
        ],
        out_specs=pl.BlockSpec(memory_space=pltpu.VMEM),
        scratch_shapes=[
            pltpu.VMEM((TY, D), jnp.float32),
            pltpu.VMEM((TY, D), jnp.bfloat16),
            pltpu.VMEM((TY, D), jnp.bfloat16),
            pltpu.SemaphoreType.DMA((C,)),
            pltpu.SemaphoreType.DMA((C,)),
            pltpu.SemaphoreType.DMA((C,)),
            pltpu.SemaphoreType.DMA((C,)),
            pltpu.SemaphoreType.DMA((C,)),
        ],
        compiler_params=pltpu.CompilerParams(collective_id=0),
    )(idxc, idx2d, E)

    return out
